# baseline (speedup 1.0000x reference)
import functools

import jax
import jax.numpy as jnp
from jax import lax
from jax.experimental import pallas as pl
from jax.experimental.pallas import tpu as pltpu
from jax.experimental.pallas import tpu_sc as plsc

B = 16384
TEXT_DIM = 256
SENT_DIM = 16
OUT_DIM = TEXT_DIM + SENT_DIM
L = 16

NUM_CORES = 2
NUM_SUBCORES = 16
NUM_WORKERS = NUM_CORES * NUM_SUBCORES
BPW = B // NUM_WORKERS
CHUNK = 64
NCHUNK = BPW // CHUNK


def _encode_body(text_hbm, ids_hbm, table_hbm, text_out_hbm, sent_hbm,
                 idx_v, table_v, sent_v, tbuf0, tbuf1, rsem0, rsem1,
                 wsem0, wsem1):
    wid = lax.axis_index("s") * NUM_CORES + lax.axis_index("c")
    base = wid * BPW

    pltpu.sync_copy(ids_hbm.at[pl.ds(base, BPW)], idx_v)
    pltpu.sync_copy(table_hbm, table_v)

    bufs = (tbuf0, tbuf1)
    rsems = (rsem0, rsem1)
    wsems = (wsem0, wsem1)
    in_cp = [None, None]
    out_cp = [None, None]
    lane = lax.iota(jnp.int32, L)

    in_cp[0] = pltpu.make_async_copy(
        text_hbm.at[pl.ds(base, CHUNK)], bufs[0], rsems[0])
    in_cp[0].start()

    for c in range(NCHUNK):
        b = c % 2
        nb = (c + 1) % 2
        if c + 1 < NCHUNK:
            if out_cp[nb] is not None:
                out_cp[nb].wait()
                out_cp[nb] = None
            in_cp[nb] = pltpu.make_async_copy(
                text_hbm.at[pl.ds(base + (c + 1) * CHUNK, CHUNK)],
                bufs[nb], rsems[nb])
            in_cp[nb].start()

        def lookup_group(p, _, _c=c):
            ids_vec = idx_v[pl.ds(_c * CHUNK + p * L, L)]
            rows = _c * CHUNK + p * L + lane
            for j in range(SENT_DIM):
                col_j = jnp.full((L,), j, jnp.int32)
                vals = plsc.load_gather(table_v, [ids_vec, col_j])
                plsc.store_scatter(sent_v, [rows, col_j], vals)
            return 0

        lax.fori_loop(0, CHUNK // L, lookup_group, 0)

        in_cp[b].wait()
        out_cp[b] = pltpu.make_async_copy(
            bufs[b], text_out_hbm.at[pl.ds(base + c * CHUNK, CHUNK)], wsems[b])
        out_cp[b].start()

    pltpu.sync_copy(sent_v, sent_hbm.at[pl.ds(base, BPW)])
    for b in range(2):
        if out_cp[b] is not None:
            out_cp[b].wait()


@functools.partial(jax.jit, static_argnames=())
def kernel(text_embed, sentiment_ids, sentiment_table):
    ids32 = sentiment_ids.astype(jnp.int32)
    mesh = plsc.VectorSubcoreMesh(core_axis_name="c", subcore_axis_name="s")
    enc = pl.kernel(
        _encode_body,
        mesh=mesh,
        compiler_params=pltpu.CompilerParams(needs_layout_passes=False),
        out_type=(jax.ShapeDtypeStruct((B, TEXT_DIM), jnp.float32),
                  jax.ShapeDtypeStruct((B, SENT_DIM), jnp.float32)),
        scratch_types=[
            pltpu.VMEM((BPW,), jnp.int32),
            pltpu.VMEM((3, SENT_DIM), jnp.float32),
            pltpu.VMEM((BPW, SENT_DIM), jnp.float32),
            pltpu.VMEM((CHUNK, TEXT_DIM), jnp.float32),
            pltpu.VMEM((CHUNK, TEXT_DIM), jnp.float32),
            pltpu.SemaphoreType.DMA,
            pltpu.SemaphoreType.DMA,
            pltpu.SemaphoreType.DMA,
            pltpu.SemaphoreType.DMA,
        ],
    )
    text_copy, sent = enc(text_embed, ids32, sentiment_table)
    return jnp.concatenate([text_copy, sent], axis=1)
